# baseline (speedup 1.0000x reference)
import functools

import jax
import jax.numpy as jnp
from jax import lax
from jax.experimental import pallas as pl
from jax.experimental.pallas import tpu as pltpu
from jax.experimental.pallas import tpu_sc as plsc

_A = 16
_B = 3
_NC = 2
_NS = 16
_NW = _NC * _NS
_GL = 16
_BK = 1024


def _sc_body(lT_hbm, wT_hbm, invt_hbm, mT_hbm, l_v, w_v, invt_v, m_v):
    rw = m_v.shape[1]
    wid = lax.axis_index("s") * _NC + lax.axis_index("c")
    base = wid * rw
    pltpu.sync_copy(lT_hbm.at[:, pl.ds(base, rw)], l_v)
    pltpu.sync_copy(wT_hbm.at[:, pl.ds(base, rw)], w_v)
    pltpu.sync_copy(invt_hbm, invt_v)
    invt = invt_v[...]

    def group(g, carry):
        row0 = g * _GL
        acc = [jnp.zeros((16,), jnp.float32) for _ in range(_A)]
        for b in range(_B):
            e = []
            for a in range(_A):
                lv = l_v[b * _A + a, pl.ds(row0, _GL)]
                e.append(jnp.exp(lv * invt))
            s = e[0]
            for a in range(1, _A):
                s = s + e[a]
            c = w_v[b, pl.ds(row0, _GL)] / s
            for a in range(_A):
                acc[a] = acc[a] + c * e[a]
        for a in range(_A):
            m_v[a, pl.ds(row0, _GL)] = acc[a]
        return carry

    lax.fori_loop(0, rw // _GL, group, 0)
    pltpu.sync_copy(m_v, mT_hbm.at[:, pl.ds(base, rw)])


def _tc_body(invt_ref, mT_ref, atoms_ref, o_ref):
    invt = invt_ref[0, 0]
    a_soft = jnp.tanh(atoms_ref[...] * invt)
    o_ref[...] = lax.dot_general(
        mT_ref[...], a_soft,
        dimension_numbers=(((0,), (0,)), ((), ())),
        preferred_element_type=jnp.float32)


@functools.partial(jax.jit, static_argnames=("interpret",))
def kernel(atoms, combo_weights, combo_indices_logits, temperature, interpret=False):
    k, b, a = combo_indices_logits.shape
    r = atoms.shape[1]
    rw = k // _NW
    invt = 1.0 / jnp.maximum(jnp.asarray(temperature, jnp.float32), 0.1)
    invt16 = jnp.broadcast_to(invt, (16,))
    lT = combo_indices_logits.transpose(1, 2, 0).reshape(b * a, k)
    wT = combo_weights.T

    mesh = plsc.VectorSubcoreMesh(core_axis_name="c", subcore_axis_name="s")
    mT = pl.kernel(
        _sc_body,
        out_type=jax.ShapeDtypeStruct((_A, k), jnp.float32),
        mesh=mesh,
        scratch_types=[
            pltpu.VMEM((b * a, rw), jnp.float32),
            pltpu.VMEM((b, rw), jnp.float32),
            pltpu.VMEM((16,), jnp.float32),
            pltpu.VMEM((_A, rw), jnp.float32),
        ],
        interpret=interpret,
    )(lT, wT, invt16)

    return pl.pallas_call(
        _tc_body,
        grid=(k // _BK,),
        in_specs=[
            pl.BlockSpec((1, 1), lambda i: (0, 0), memory_space=pltpu.SMEM),
            pl.BlockSpec((_A, _BK), lambda i: (0, i)),
            pl.BlockSpec((a, r), lambda i: (0, 0)),
        ],
        out_specs=pl.BlockSpec((_BK, r), lambda i: (i, 0)),
        out_shape=jax.ShapeDtypeStruct((k, r), jnp.float32),
        interpret=interpret,
    )(invt.reshape(1, 1), mT, atoms)


# speedup vs baseline: 2.3182x; 2.3182x over previous
import functools

import jax
import jax.numpy as jnp
from jax import lax
from jax.experimental import pallas as pl
from jax.experimental.pallas import tpu as pltpu

_A = 16
_B = 3
_BK = 512


def _body(invt_ref, lT_ref, wT_ref, atoms_ref, o_ref):
    invt = invt_ref[0, 0]
    e = jnp.exp(lT_ref[...] * invt)
    e3 = e.reshape(_B, _A, e.shape[-1])
    s = jnp.sum(e3, axis=1, keepdims=True)
    c = wT_ref[...].reshape(_B, 1, -1) / s
    m = jnp.sum(e3 * c, axis=0)
    a_soft = jnp.tanh(atoms_ref[...] * invt)
    o_ref[...] = lax.dot_general(
        m, a_soft, dimension_numbers=(((0,), (0,)), ((), ())),
        preferred_element_type=jnp.float32)


@functools.partial(jax.jit, static_argnames=("interpret",))
def kernel(atoms, combo_weights, combo_indices_logits, temperature, interpret=False):
    k, b, a = combo_indices_logits.shape
    r = atoms.shape[1]
    invt = (1.0 / jnp.maximum(jnp.asarray(temperature, jnp.float32), 0.1))
    invt = invt.reshape(1, 1)
    lT = combo_indices_logits.transpose(1, 2, 0).reshape(b * a, k)
    wT = combo_weights.T
    grid = (k // _BK,)
    return pl.pallas_call(
        _body,
        grid=grid,
        in_specs=[
            pl.BlockSpec((1, 1), lambda i: (0, 0), memory_space=pltpu.SMEM),
            pl.BlockSpec((b * a, _BK), lambda i: (0, i)),
            pl.BlockSpec((b, _BK), lambda i: (0, i)),
            pl.BlockSpec((a, r), lambda i: (0, 0)),
        ],
        out_specs=pl.BlockSpec((_BK, r), lambda i: (i, 0)),
        out_shape=jax.ShapeDtypeStruct((k, r), jnp.float32),
        interpret=interpret,
    )(invt, lT, wT, atoms)


# speedup vs baseline: 4.4635x; 1.9254x over previous
import functools

import jax
import jax.numpy as jnp
from jax import lax
from jax.experimental import pallas as pl
from jax.experimental.pallas import tpu as pltpu

_A = 16
_B = 3
_BK = 2048


def _body(invt_ref, lT_ref, wT_ref, atoms_ref, o_ref):
    invt = invt_ref[0, 0]
    e = jnp.exp(lT_ref[...] * invt)
    e3 = e.reshape(_B, _A, e.shape[-1])
    s = jnp.sum(e3, axis=1, keepdims=True)
    c = wT_ref[...].reshape(_B, 1, -1) / s
    m = jnp.sum(e3 * c, axis=0)
    a_soft = jnp.tanh(atoms_ref[...] * invt)
    o_ref[...] = lax.dot_general(
        m, a_soft, dimension_numbers=(((0,), (0,)), ((), ())),
        preferred_element_type=jnp.float32)


@functools.partial(jax.jit, static_argnames=("interpret",))
def kernel(atoms, combo_weights, combo_indices_logits, temperature, interpret=False):
    k, b, a = combo_indices_logits.shape
    r = atoms.shape[1]
    invt = (1.0 / jnp.maximum(jnp.asarray(temperature, jnp.float32), 0.1))
    invt = invt.reshape(1, 1)
    lT = combo_indices_logits.transpose(1, 2, 0).reshape(b * a, k)
    wT = combo_weights.T
    grid = (k // _BK,)
    return pl.pallas_call(
        _body,
        grid=grid,
        in_specs=[
            pl.BlockSpec((1, 1), lambda i: (0, 0), memory_space=pltpu.SMEM),
            pl.BlockSpec((b * a, _BK), lambda i: (0, i)),
            pl.BlockSpec((b, _BK), lambda i: (0, i)),
            pl.BlockSpec((a, r), lambda i: (0, 0)),
        ],
        out_specs=pl.BlockSpec((_BK, r), lambda i: (i, 0)),
        out_shape=jax.ShapeDtypeStruct((k, r), jnp.float32),
        interpret=interpret,
    )(invt, lT, wT, atoms)


# speedup vs baseline: 5.1403x; 1.1516x over previous
import functools

import jax
import jax.numpy as jnp
from jax import lax
from jax.experimental import pallas as pl
from jax.experimental.pallas import tpu as pltpu

_A = 16
_B = 3
_BK = 4096


def _body(invt_ref, lT_ref, wT_ref, atoms_ref, o_ref):
    invt = invt_ref[0, 0]
    e = jnp.exp(lT_ref[...] * invt)
    e3 = e.reshape(_B, _A, e.shape[-1])
    s = jnp.sum(e3, axis=1, keepdims=True)
    c = wT_ref[...].reshape(_B, 1, -1) / s
    m = jnp.sum(e3 * c, axis=0)
    a_soft = jnp.tanh(atoms_ref[...] * invt)
    o_ref[...] = lax.dot_general(
        m, a_soft, dimension_numbers=(((0,), (0,)), ((), ())),
        preferred_element_type=jnp.float32)


@functools.partial(jax.jit, static_argnames=("interpret",))
def kernel(atoms, combo_weights, combo_indices_logits, temperature, interpret=False):
    k, b, a = combo_indices_logits.shape
    r = atoms.shape[1]
    invt = (1.0 / jnp.maximum(jnp.asarray(temperature, jnp.float32), 0.1))
    invt = invt.reshape(1, 1)
    lT = combo_indices_logits.transpose(1, 2, 0).reshape(b * a, k)
    wT = combo_weights.T
    grid = (k // _BK,)
    return pl.pallas_call(
        _body,
        grid=grid,
        in_specs=[
            pl.BlockSpec((1, 1), lambda i: (0, 0), memory_space=pltpu.SMEM),
            pl.BlockSpec((b * a, _BK), lambda i: (0, i)),
            pl.BlockSpec((b, _BK), lambda i: (0, i)),
            pl.BlockSpec((a, r), lambda i: (0, 0)),
        ],
        out_specs=pl.BlockSpec((_BK, r), lambda i: (i, 0)),
        out_shape=jax.ShapeDtypeStruct((k, r), jnp.float32),
        interpret=interpret,
    )(invt, lT, wT, atoms)
